# baseline (speedup 1.0000x reference)
import functools

import jax
import jax.numpy as jnp
from jax import lax
from jax.experimental import pallas as pl
from jax.experimental.pallas import tpu as pltpu
from jax.experimental.pallas import tpu_sc as plsc

VOCAB_SIZE = 1000000
EMBED_SIZE = 64
BATCH = 16384
HIST_LEN = 50

NC, NS = 2, 16
NW = NC * NS
BW = BATCH // NW
CHUNK = 256
NSLOT = 2


def _make_kernel():
  mesh = plsc.VectorSubcoreMesh(core_axis_name="c", subcore_axis_name="s")

  @functools.partial(
      pl.kernel,
      mesh=mesh,
      out_type=jax.ShapeDtypeStruct((HIST_LEN, EMBED_SIZE, BATCH),
                                    jnp.float32),
      scratch_types=[
          pltpu.VMEM((HIST_LEN, BW), jnp.int32),
          pltpu.VMEM((NSLOT, CHUNK, EMBED_SIZE), jnp.float32),
          pltpu.VMEM((NSLOT, EMBED_SIZE, CHUNK), jnp.float32),
          pltpu.SemaphoreType.DMA,
          pltpu.SemaphoreType.DMA((NSLOT,)),
          pltpu.SemaphoreType.DMA((NSLOT,)),
      ],
      compiler_params=pltpu.CompilerParams(
          use_tc_tiling_on_sc=False, needs_layout_passes=False),
  )
  def emb(idx_hbm, table_hbm, out_hbm, idx_all, rows, rowsT, isem, gsem,
          wsem):
    wid = lax.axis_index("s") * NC + lax.axis_index("c")
    b0 = wid * BW

    for h in range(HIST_LEN):
      pltpu.make_async_copy(
          idx_hbm.at[pl.ds(h * BATCH + b0, BW)], idx_all.at[h], isem
      ).start()
    for h in range(HIST_LEN):
      pltpu.make_async_copy(
          idx_hbm.at[pl.ds(h * BATCH + b0, BW)], idx_all.at[h], isem
      ).wait()

    def gather_copy(h, s):
      return pltpu.make_async_copy(
          table_hbm.at[idx_all.at[h, pl.ds(s * CHUNK, CHUNK)]],
          rows.at[s], gsem.at[s])

    def wb_copy(h, s):
      return pltpu.make_async_copy(
          rowsT.at[s], out_hbm.at[h, :, pl.ds(b0 + s * CHUNK, CHUNK)],
          wsem.at[s])

    row_ids = [lax.iota(jnp.int32, 16) + 16 * q for q in range(4)]

    def transpose_chunk(s):
      def body_c(cb, carry):
        c0 = cb * 8
        for dc in range(8):
          c = c0 + dc
          col = jnp.full((16,), 0, jnp.int32) + c
          for q in range(4):
            v = rows[s, c, pl.ds(16 * q, 16)]
            plsc.store_scatter(rowsT.at[s], [row_ids[q], col], v)
        return carry
      lax.fori_loop(0, CHUNK // 8, body_c, 0)

    for s in range(NSLOT):
      gather_copy(0, s).start()

    def round_fn(h, first, last):
      for s in range(NSLOT):
        if not first:
          wb_copy(h - 1, s).wait()
        gather_copy(h, s).wait()
        transpose_chunk(s)
        wb_copy(h, s).start()
        if not last:
          gather_copy(h + 1, s).start()

    round_fn(0, True, False)

    def body(h, carry):
      round_fn(h, False, False)
      return carry

    lax.fori_loop(1, HIST_LEN - 1, body, 0)
    round_fn(HIST_LEN - 1, False, True)
    for s in range(NSLOT):
      wb_copy(HIST_LEN - 1, s).wait()

  return emb


_emb = _make_kernel()


@jax.jit
def kernel(x, table):
  idx = x.T.reshape(-1).astype(jnp.int32)
  out = _emb(idx, table)
  return out.transpose(2, 0, 1)


# speedup vs baseline: 1.1896x; 1.1896x over previous
import functools

import jax
import jax.numpy as jnp
from jax import lax
from jax.experimental import pallas as pl
from jax.experimental.pallas import tpu as pltpu
from jax.experimental.pallas import tpu_sc as plsc

VOCAB_SIZE = 1000000
EMBED_SIZE = 64
BATCH = 16384
HIST_LEN = 50

NC, NS = 2, 16
NW = NC * NS
BW = BATCH // NW
CHUNK = 256
NSLOT = 2


def _make_kernel():
  mesh = plsc.VectorSubcoreMesh(core_axis_name="c", subcore_axis_name="s")

  @functools.partial(
      pl.kernel,
      mesh=mesh,
      out_type=jax.ShapeDtypeStruct((HIST_LEN, EMBED_SIZE, BATCH),
                                    jnp.float32),
      scratch_types=[
          pltpu.VMEM((HIST_LEN, BW), jnp.int32),
          pltpu.VMEM((NSLOT, CHUNK, EMBED_SIZE), jnp.float32),
          pltpu.VMEM((NSLOT, EMBED_SIZE, CHUNK), jnp.float32),
          pltpu.SemaphoreType.DMA,
          pltpu.SemaphoreType.DMA((NSLOT,)),
          pltpu.SemaphoreType.DMA((NSLOT,)),
      ],
      compiler_params=pltpu.CompilerParams(
          use_tc_tiling_on_sc=False, needs_layout_passes=False),
  )
  def emb(idx_hbm, table_hbm, out_hbm, idx_all, rows, rowsT, isem, gsem,
          wsem):
    wid = lax.axis_index("s") * NC + lax.axis_index("c")
    b0 = wid * BW

    for h in range(HIST_LEN):
      pltpu.make_async_copy(
          idx_hbm.at[pl.ds(h * BATCH + b0, BW)], idx_all.at[h], isem
      ).start()
    for h in range(HIST_LEN):
      pltpu.make_async_copy(
          idx_hbm.at[pl.ds(h * BATCH + b0, BW)], idx_all.at[h], isem
      ).wait()

    def gather_copy(h, s):
      return pltpu.make_async_copy(
          table_hbm.at[idx_all.at[h, pl.ds(s * CHUNK, CHUNK)]],
          rows.at[s], gsem.at[s])

    def wb_copy(h, s):
      return pltpu.make_async_copy(
          rowsT.at[s], out_hbm.at[h, :, pl.ds(b0 + s * CHUNK, CHUNK)],
          wsem.at[s])

    row_ids = [lax.iota(jnp.int32, 16) + 16 * q for q in range(4)]

    def transpose_chunk(s):
      @plsc.parallel_loop(0, CHUNK, step=1, unroll=8)
      def body_c(c):
        col = jnp.full((16,), 0, jnp.int32) + c
        for q in range(4):
          v = rows[s, c, pl.ds(16 * q, 16)]
          plsc.store_scatter(rowsT.at[s], [row_ids[q], col], v)

    for s in range(NSLOT):
      gather_copy(0, s).start()

    def round_fn(h, first, last):
      for s in range(NSLOT):
        if not first:
          wb_copy(h - 1, s).wait()
        gather_copy(h, s).wait()
        transpose_chunk(s)
        wb_copy(h, s).start()
        if not last:
          gather_copy(h + 1, s).start()

    round_fn(0, True, False)

    def body(h, carry):
      round_fn(h, False, False)
      return carry

    lax.fori_loop(1, HIST_LEN - 1, body, 0)
    round_fn(HIST_LEN - 1, False, True)
    for s in range(NSLOT):
      wb_copy(HIST_LEN - 1, s).wait()

  return emb


_emb = _make_kernel()


@jax.jit
def kernel(x, table):
  idx = x.T.reshape(-1).astype(jnp.int32)
  out = _emb(idx, table)
  return out.transpose(2, 0, 1)


# speedup vs baseline: 1.5100x; 1.2693x over previous
import functools

import jax
import jax.numpy as jnp
from jax import lax
from jax.experimental import pallas as pl
from jax.experimental.pallas import tpu as pltpu
from jax.experimental.pallas import tpu_sc as plsc

VOCAB_SIZE = 1000000
EMBED_SIZE = 64
BATCH = 16384
HIST_LEN = 50

B = BATCH * HIST_LEN
NC, NS = 2, 16
NW = NC * NS
B_PER_W = B // NW
CHUNK = 256
NCHUNK = B_PER_W // CHUNK
NBUF = 4
NITER = NCHUNK // NBUF


def _make_kernel():
  mesh = plsc.VectorSubcoreMesh(core_axis_name="c", subcore_axis_name="s")

  @functools.partial(
      pl.kernel,
      mesh=mesh,
      out_type=jax.ShapeDtypeStruct((B, EMBED_SIZE), jnp.float32),
      scratch_types=[
          pltpu.VMEM((B_PER_W,), jnp.int32),
          pltpu.VMEM((NBUF, CHUNK, EMBED_SIZE), jnp.float32),
          pltpu.SemaphoreType.DMA((NBUF,)),
          pltpu.SemaphoreType.DMA((NBUF,)),
      ],
      compiler_params=pltpu.CompilerParams(use_tc_tiling_on_sc=False),
  )
  def emb(idx_hbm, table_hbm, out_hbm, idx_all, rows_v, gsem, osem):
    wid = lax.axis_index("s") * NC + lax.axis_index("c")
    base = wid * B_PER_W
    pltpu.sync_copy(idx_hbm.at[pl.ds(base, B_PER_W)], idx_all)

    def gather_copy(g, b):
      return pltpu.make_async_copy(
          table_hbm.at[idx_all.at[pl.ds(g * CHUNK, CHUNK)]],
          rows_v.at[b], gsem.at[b])

    def wb_copy(g, b):
      return pltpu.make_async_copy(
          rows_v.at[b], out_hbm.at[pl.ds(base + g * CHUNK, CHUNK)],
          osem.at[b])

    for b in range(NBUF):
      gather_copy(b, b).start()

    def round_fn(i, start_next):
      g0 = i * NBUF
      for b in range(NBUF):
        gather_copy(g0 + b, b).wait()
        wb_copy(g0 + b, b).start()
      for b in range(NBUF):
        wb_copy(g0 + b, b).wait()
        if start_next:
          gather_copy(g0 + NBUF + b, b).start()

    def body(i, carry):
      round_fn(i, True)
      return carry

    lax.fori_loop(0, NITER - 1, body, 0)
    round_fn(NITER - 1, False)

  return emb


_emb = _make_kernel()


@jax.jit
def kernel(x, table):
  idx = x.T.reshape(-1).astype(jnp.int32)
  out = _emb(idx, table)
  return out.reshape(HIST_LEN, BATCH, EMBED_SIZE).transpose(1, 0, 2)
